# baseline (speedup 1.0000x reference)
import functools

import jax
import jax.numpy as jnp
from jax import lax
from jax.experimental import pallas as pl
from jax.experimental.pallas import tpu as pltpu

F = 40960
B = 1024
BK = 1024
NK = F // BK


def _l1_body(x1_ref, x2_ref, w1_ref, h1_ref, h2_ref):
    k = pl.program_id(0)
    wb = w1_ref[...].astype(jnp.bfloat16)
    x1b = x1_ref[...].astype(jnp.bfloat16)
    x2b = x2_ref[...].astype(jnp.bfloat16)
    dn = (((1,), (1,)), ((), ()))
    a1 = lax.dot_general(x1b, wb, dn, preferred_element_type=jnp.float32)
    a2 = lax.dot_general(x2b, wb, dn, preferred_element_type=jnp.float32)

    @pl.when(k == 0)
    def _():
        h1_ref[...] = a1
        h2_ref[...] = a2

    @pl.when(k > 0)
    def _():
        h1_ref[...] += a1
        h2_ref[...] += a2


def _ln_lrelu(v):
    mu = jnp.mean(v, axis=1, keepdims=True)
    var = jnp.mean((v - mu) ** 2, axis=1, keepdims=True)
    y = (v - mu) * lax.rsqrt(var)
    return jnp.maximum(0.05 * y, y)


def _mlp_body(h1_ref, h2_ref, w2_ref, w3_ref, w4_ref, out_ref):
    dn = (((1,), (1,)), ((), ()))
    g1 = _ln_lrelu(h1_ref[...])
    g2 = _ln_lrelu(h2_ref[...])
    w2 = w2_ref[...]
    m1 = _ln_lrelu(lax.dot_general(g1, w2, dn, preferred_element_type=jnp.float32))
    m2 = _ln_lrelu(lax.dot_general(g2, w2, dn, preferred_element_type=jnp.float32))
    w3a = w3_ref[:, :64]
    w3b = w3_ref[:, 64:]
    s = (lax.dot_general(m1, w3a, dn, preferred_element_type=jnp.float32)
         + lax.dot_general(m2, w3b, dn, preferred_element_type=jnp.float32))
    s = _ln_lrelu(s)
    out_ref[...] = lax.dot_general(s, w4_ref[...], dn,
                                   preferred_element_type=jnp.float32)


@jax.jit
def kernel(x, W1, W2, W3, W4):
    h1, h2 = pl.pallas_call(
        _l1_body,
        grid=(NK,),
        in_specs=[
            pl.BlockSpec((B, BK), lambda k: (0, k)),
            pl.BlockSpec((B, BK), lambda k: (0, NK + k)),
            pl.BlockSpec((256, BK), lambda k: (0, k)),
        ],
        out_specs=[
            pl.BlockSpec((B, 256), lambda k: (0, 0)),
            pl.BlockSpec((B, 256), lambda k: (0, 0)),
        ],
        out_shape=[
            jax.ShapeDtypeStruct((B, 256), jnp.float32),
            jax.ShapeDtypeStruct((B, 256), jnp.float32),
        ],
    )(x, x, W1)

    out = pl.pallas_call(
        _mlp_body,
        out_shape=jax.ShapeDtypeStruct((B, 1), jnp.float32),
    )(h1, h2, W2, W3, W4)
    return out


# speedup vs baseline: 1.0332x; 1.0332x over previous
import jax
import jax.numpy as jnp
from jax import lax
from jax.experimental import pallas as pl
from jax.experimental.pallas import tpu as pltpu

F = 40960
B = 1024
BK = 2048
NK = F // BK


def _ln_lrelu(v):
    mu = jnp.mean(v, axis=1, keepdims=True)
    var = jnp.mean((v - mu) ** 2, axis=1, keepdims=True)
    y = (v - mu) * lax.rsqrt(var)
    return jnp.maximum(0.05 * y, y)


def _body(x1_ref, x2_ref, w1_ref, w2_ref, w3_ref, w4_ref, out_ref,
          h1_s, h2_s):
    k = pl.program_id(0)
    dn = (((1,), (1,)), ((), ()))
    wb = w1_ref[...].astype(jnp.bfloat16)
    x1b = x1_ref[...].astype(jnp.bfloat16)
    x2b = x2_ref[...].astype(jnp.bfloat16)
    a1 = lax.dot_general(x1b, wb, dn, preferred_element_type=jnp.float32)
    a2 = lax.dot_general(x2b, wb, dn, preferred_element_type=jnp.float32)

    @pl.when(k == 0)
    def _():
        h1_s[...] = a1
        h2_s[...] = a2

    @pl.when(k > 0)
    def _():
        h1_s[...] += a1
        h2_s[...] += a2

    @pl.when(k == NK - 1)
    def _():
        g1 = _ln_lrelu(h1_s[...])
        g2 = _ln_lrelu(h2_s[...])
        w2 = w2_ref[...]
        m1 = _ln_lrelu(lax.dot_general(g1, w2, dn,
                                       preferred_element_type=jnp.float32))
        m2 = _ln_lrelu(lax.dot_general(g2, w2, dn,
                                       preferred_element_type=jnp.float32))
        w3a = w3_ref[:, :64]
        w3b = w3_ref[:, 64:]
        s = (lax.dot_general(m1, w3a, dn, preferred_element_type=jnp.float32)
             + lax.dot_general(m2, w3b, dn,
                               preferred_element_type=jnp.float32))
        s = _ln_lrelu(s)
        out_ref[...] = lax.dot_general(s, w4_ref[...], dn,
                                       preferred_element_type=jnp.float32)


@jax.jit
def kernel(x, W1, W2, W3, W4):
    return pl.pallas_call(
        _body,
        grid=(NK,),
        in_specs=[
            pl.BlockSpec((B, BK), lambda k: (0, k)),
            pl.BlockSpec((B, BK), lambda k: (0, NK + k)),
            pl.BlockSpec((256, BK), lambda k: (0, k)),
            pl.BlockSpec((64, 256), lambda k: (0, 0)),
            pl.BlockSpec((8, 128), lambda k: (0, 0)),
            pl.BlockSpec((1, 8), lambda k: (0, 0)),
        ],
        out_specs=pl.BlockSpec((B, 1), lambda k: (0, 0)),
        out_shape=jax.ShapeDtypeStruct((B, 1), jnp.float32),
        scratch_shapes=[
            pltpu.VMEM((B, 256), jnp.float32),
            pltpu.VMEM((B, 256), jnp.float32),
        ],
    )(x, x, W1, W2, W3, W4)
